# baseline (speedup 1.0000x reference)
import jax
import jax.numpy as jnp
from jax.experimental import pallas as pl
from jax.experimental.pallas import tpu as pltpu

_DOT_DIMS = (((1,), (0,)), ((), ()))


def _xw_kernel(x_ref, w_ref, o_ref):
    o_ref[...] = jax.lax.dot_general(
        x_ref[...], w_ref[...], _DOT_DIMS, preferred_element_type=jnp.float32)


def _fused1_kernel(a_ref, s1_ref, b1_ref, w2_ref, o_ref):
    h = jax.lax.dot_general(
        a_ref[...], s1_ref[...], _DOT_DIMS, preferred_element_type=jnp.float32)
    h = jnp.maximum(h + b1_ref[...], 0.0)
    o_ref[...] = jax.lax.dot_general(
        h, w2_ref[...], _DOT_DIMS, preferred_element_type=jnp.float32)


def _pass2_kernel(a_ref, g_ref, b2_ref, o_ref):
    o_ref[...] = jax.lax.dot_general(
        a_ref[...], g_ref[...], _DOT_DIMS,
        preferred_element_type=jnp.float32) + b2_ref[...]


def kernel(features, matrix_sparse, W1, b1, W2, b2):
    n, d = features.shape
    h1 = W1.shape[1]
    h2 = W2.shape[1]
    b1r = b1.reshape(1, h1)
    b2r = b2.reshape(1, h2)

    bmx = 2000
    s1 = pl.pallas_call(
        _xw_kernel,
        grid=(n // bmx,),
        in_specs=[
            pl.BlockSpec((bmx, d), lambda i: (i, 0)),
            pl.BlockSpec((d, h1), lambda i: (0, 0)),
        ],
        out_specs=pl.BlockSpec((bmx, h1), lambda i: (i, 0)),
        out_shape=jax.ShapeDtypeStruct((n, h1), jnp.float32),
        compiler_params=pltpu.CompilerParams(
            dimension_semantics=("parallel",)),
    )(features, W1)

    bm = 400
    g = pl.pallas_call(
        _fused1_kernel,
        grid=(n // bm,),
        in_specs=[
            pl.BlockSpec((bm, n), lambda i: (i, 0)),
            pl.BlockSpec((n, h1), lambda i: (0, 0)),
            pl.BlockSpec((1, h1), lambda i: (0, 0)),
            pl.BlockSpec((h1, h2), lambda i: (0, 0)),
        ],
        out_specs=pl.BlockSpec((bm, h2), lambda i: (i, 0)),
        out_shape=jax.ShapeDtypeStruct((n, h2), jnp.float32),
        compiler_params=pltpu.CompilerParams(
            dimension_semantics=("parallel",)),
    )(matrix_sparse, s1, b1r, W2)

    out = pl.pallas_call(
        _pass2_kernel,
        grid=(n // bm,),
        in_specs=[
            pl.BlockSpec((bm, n), lambda i: (i, 0)),
            pl.BlockSpec((n, h2), lambda i: (0, 0)),
            pl.BlockSpec((1, h2), lambda i: (0, 0)),
        ],
        out_specs=pl.BlockSpec((bm, h2), lambda i: (i, 0)),
        out_shape=jax.ShapeDtypeStruct((n, h2), jnp.float32),
        compiler_params=pltpu.CompilerParams(
            dimension_semantics=("parallel",)),
    )(matrix_sparse, g, b2r)

    return out


# speedup vs baseline: 1.0016x; 1.0016x over previous
import jax
import jax.numpy as jnp
from jax.experimental import pallas as pl
from jax.experimental.pallas import tpu as pltpu

_DOT_DIMS = (((1,), (0,)), ((), ()))


def _xw_kernel(x_ref, w_ref, o_ref):
    o_ref[...] = jax.lax.dot_general(
        x_ref[...], w_ref[...], _DOT_DIMS, preferred_element_type=jnp.float32)


def _fused1_kernel(a_ref, s1_ref, b1_ref, w2_ref, o_ref):
    h = jax.lax.dot_general(
        a_ref[...].astype(jnp.bfloat16), s1_ref[...].astype(jnp.bfloat16),
        _DOT_DIMS, preferred_element_type=jnp.float32)
    h = jnp.maximum(h + b1_ref[...], 0.0)
    o_ref[...] = jax.lax.dot_general(
        h, w2_ref[...], _DOT_DIMS, preferred_element_type=jnp.float32)


def _pass2_kernel(a_ref, g_ref, b2_ref, o_ref):
    o_ref[...] = jax.lax.dot_general(
        a_ref[...].astype(jnp.bfloat16), g_ref[...].astype(jnp.bfloat16),
        _DOT_DIMS, preferred_element_type=jnp.float32) + b2_ref[...]


def kernel(features, matrix_sparse, W1, b1, W2, b2):
    n, d = features.shape
    h1 = W1.shape[1]
    h2 = W2.shape[1]
    b1r = b1.reshape(1, h1)
    b2r = b2.reshape(1, h2)

    bmx = 2000
    s1 = pl.pallas_call(
        _xw_kernel,
        grid=(n // bmx,),
        in_specs=[
            pl.BlockSpec((bmx, d), lambda i: (i, 0)),
            pl.BlockSpec((d, h1), lambda i: (0, 0)),
        ],
        out_specs=pl.BlockSpec((bmx, h1), lambda i: (i, 0)),
        out_shape=jax.ShapeDtypeStruct((n, h1), jnp.float32),
        compiler_params=pltpu.CompilerParams(
            dimension_semantics=("parallel",)),
    )(features, W1)

    bm = 400
    g = pl.pallas_call(
        _fused1_kernel,
        grid=(n // bm,),
        in_specs=[
            pl.BlockSpec((bm, n), lambda i: (i, 0)),
            pl.BlockSpec((n, h1), lambda i: (0, 0)),
            pl.BlockSpec((1, h1), lambda i: (0, 0)),
            pl.BlockSpec((h1, h2), lambda i: (0, 0)),
        ],
        out_specs=pl.BlockSpec((bm, h2), lambda i: (i, 0)),
        out_shape=jax.ShapeDtypeStruct((n, h2), jnp.float32),
        compiler_params=pltpu.CompilerParams(
            dimension_semantics=("parallel",)),
    )(matrix_sparse, s1, b1r, W2)

    out = pl.pallas_call(
        _pass2_kernel,
        grid=(n // bm,),
        in_specs=[
            pl.BlockSpec((bm, n), lambda i: (i, 0)),
            pl.BlockSpec((n, h2), lambda i: (0, 0)),
            pl.BlockSpec((1, h2), lambda i: (0, 0)),
        ],
        out_specs=pl.BlockSpec((bm, h2), lambda i: (i, 0)),
        out_shape=jax.ShapeDtypeStruct((n, h2), jnp.float32),
        compiler_params=pltpu.CompilerParams(
            dimension_semantics=("parallel",)),
    )(matrix_sparse, g, b2r)

    return out
